# baseline (speedup 1.0000x reference)
import functools

import jax
import jax.numpy as jnp
import numpy as np
from jax.experimental import pallas as pl
from jax.experimental.pallas import tpu as pltpu

_KEY_HI = np.uint32(0)
_KEY_LO = np.uint32(1234)
_INT_MAX = 2**31 - 1


def _rotl(x, r):
    return (x << np.uint32(r)) | (x >> np.uint32(32 - r))


def _threefry_bits(lo):
    ks0 = _KEY_HI
    ks1 = _KEY_LO
    ks2 = np.uint32(0x1BD11BDA) ^ ks0 ^ ks1
    ks = (ks0, ks1, ks2)
    rotations = ((13, 15, 26, 6), (17, 29, 16, 24))
    x0 = jnp.zeros_like(lo) + ks0
    x1 = lo + ks1
    for i in range(5):
        for r in rotations[i % 2]:
            x0 = x0 + x1
            x1 = _rotl(x1, r)
            x1 = x1 ^ x0
        x0 = x0 + ks[(i + 1) % 3]
        x1 = x1 + ks[(i + 2) % 3] + np.uint32(i + 1)
    return x0 ^ x1


def _bits_to_gumbel(bits):
    fb = (bits >> np.uint32(9)) | np.uint32(0x3F800000)
    u = jax.lax.bitcast_convert_type(fb, jnp.float32) - np.float32(1.0)
    tiny = np.float32(np.finfo(np.float32).tiny)
    u = jnp.maximum(tiny, u * (np.float32(1.0) - tiny) + tiny)
    return -jnp.log(-jnp.log(u))


def _tile_kernel(x_ref, w_ref, b_ref, tmax_ref, targ_ref, *, tn, vocab):
    j = pl.program_id(0)
    m = x_ref.shape[0]
    logits = jnp.dot(x_ref[...], w_ref[...], preferred_element_type=jnp.float32)
    col = jax.lax.broadcasted_iota(jnp.int32, (m, tn), 1) + j * tn
    row = jax.lax.broadcasted_iota(jnp.uint32, (m, tn), 0)
    lo = row * np.uint32(vocab) + col.astype(jnp.uint32)
    g = _bits_to_gumbel(_threefry_bits(lo))
    vals = logits + b_ref[...] + g
    vals = jnp.where(col < vocab, vals, -jnp.inf)
    local_max = jnp.max(vals, axis=1, keepdims=True)
    cand = jnp.where(vals == local_max, col, _INT_MAX)
    tmax_ref[...] = local_max.reshape(1, m, 1)
    targ_ref[...] = jnp.min(cand, axis=1, keepdims=True).reshape(1, m, 1)


def _reduce_kernel(tmax_ref, targ_ref, idx_ref):
    t = tmax_ref[...]
    a = targ_ref[...]
    row_max = jnp.max(t, axis=0, keepdims=True)
    cand = jnp.where(t == row_max, a, _INT_MAX)
    idx_ref[...] = jnp.min(cand, axis=0)


def _onehot_kernel(idx_ref, out_ref, *, tn):
    j = pl.program_id(0)
    m = out_ref.shape[0]
    col = jax.lax.broadcasted_iota(jnp.int32, (m, tn), 1) + j * tn
    out_ref[...] = (col == idx_ref[...]).astype(jnp.float32)


def _run(x, W, b, *, tn, tn2):
    m, k = x.shape
    vocab = W.shape[1]
    nt = pl.cdiv(vocab, tn)
    b2 = b.reshape(1, vocab)
    tmax, targ = pl.pallas_call(
        functools.partial(_tile_kernel, tn=tn, vocab=vocab),
        grid=(nt,),
        in_specs=[
            pl.BlockSpec((m, k), lambda j: (0, 0)),
            pl.BlockSpec((k, tn), lambda j: (0, j)),
            pl.BlockSpec((1, tn), lambda j: (0, j)),
        ],
        out_specs=(
            pl.BlockSpec((1, m, 1), lambda j: (j, 0, 0)),
            pl.BlockSpec((1, m, 1), lambda j: (j, 0, 0)),
        ),
        out_shape=(
            jax.ShapeDtypeStruct((nt, m, 1), jnp.float32),
            jax.ShapeDtypeStruct((nt, m, 1), jnp.int32),
        ),
        compiler_params=pltpu.CompilerParams(
            dimension_semantics=("parallel",),
        ),
    )(x, W, b2)
    idx = pl.pallas_call(
        _reduce_kernel,
        grid=(1,),
        in_specs=[
            pl.BlockSpec((nt, m, 1), lambda j: (0, 0, 0)),
            pl.BlockSpec((nt, m, 1), lambda j: (0, 0, 0)),
        ],
        out_specs=pl.BlockSpec((m, 1), lambda j: (0, 0)),
        out_shape=jax.ShapeDtypeStruct((m, 1), jnp.int32),
    )(tmax, targ)
    nt2 = pl.cdiv(vocab, tn2)
    out = pl.pallas_call(
        functools.partial(_onehot_kernel, tn=tn2),
        grid=(nt2,),
        in_specs=[pl.BlockSpec((m, 1), lambda j: (0, 0))],
        out_specs=pl.BlockSpec((m, tn2), lambda j: (0, j)),
        out_shape=jax.ShapeDtypeStruct((m, vocab), jnp.float32),
        compiler_params=pltpu.CompilerParams(
            dimension_semantics=("parallel",),
        ),
    )(idx)
    return out


def kernel(x, W, b):
    return _run(x, W, b, tn=1024, tn2=4096)
